# Initial kernel scaffold; baseline (speedup 1.0000x reference)
import functools

import jax
import jax.numpy as jnp
from jax import lax
from jax.experimental import pallas as pl
from jax.experimental.pallas import tpu as pltpu
from jax.experimental.pallas import tpu_sc as plsc

N = 10000
E = 320000
IN = 128
HID = 256
OUT = 128

BLK = 128
R = 10112
WE = 144
NW = 32
CHUNK = 128
NCHUNK = 79
EW = NCHUNK * CHUNK
EPAD = NW * EW
RPT = R // 16


def _make_sc_agg(width):
    mesh = plsc.VectorSubcoreMesh(core_axis_name="c", subcore_axis_name="s")

    @functools.partial(
        pl.kernel,
        mesh=mesh,
        out_type=jax.ShapeDtypeStruct((2, R, width), jnp.float32),
        scratch_types=[
            pltpu.VMEM((NCHUNK, CHUNK), jnp.int32),
            pltpu.VMEM((NCHUNK, CHUNK), jnp.int32),
            pltpu.VMEM((CHUNK, width), jnp.float32),
            pltpu.VMEM_SHARED((R, width), jnp.float32),
            pltpu.SemaphoreType.DMA,
        ],
    )
    def sc_agg(rows_hbm, src_hbm, dst_hbm, zeros_hbm, out_hbm,
               src_v, dst_v, rows_v, acc, sem):
        c = lax.axis_index("c")
        s = lax.axis_index("s")
        wid = c * 16 + s
        pltpu.sync_copy(src_hbm.at[wid], src_v)
        pltpu.sync_copy(dst_hbm.at[wid], dst_v)
        pltpu.sync_copy(zeros_hbm.at[pl.ds(s * RPT, RPT)],
                        acc.at[pl.ds(s * RPT, RPT)])
        plsc.subcore_barrier()

        def body(i, carry):
            pltpu.async_copy(rows_hbm.at[src_v.at[i]], rows_v, sem).wait()
            pltpu.sync_copy(rows_v, acc.at[dst_v.at[i]], add=True)
            return carry

        lax.fori_loop(0, NCHUNK, body, 0)
        plsc.subcore_barrier()
        pltpu.sync_copy(acc.at[pl.ds(s * RPT, RPT)],
                        out_hbm.at[c, pl.ds(s * RPT, RPT)])

    return sc_agg


_sc_agg_l1 = _make_sc_agg(WE)
_sc_agg_l2 = _make_sc_agg(OUT)


def _dot_t(a, b):
    return lax.dot_general(a, b, (((1,), (1,)), ((), ())),
                           preferred_element_type=jnp.float32)


def _tc1_body(p0, p1, xb, w1l, b1l, w1r, w2l, h_out, p2_out, dinv_out):
    agg = p0[...] + p1[...]
    a = agg[:, :IN]
    deg = agg[:, IN:IN + 1]
    dinv = 1.0 / jnp.maximum(deg, 1.0)
    mean = a * dinv
    h = _dot_t(mean, w1l[...]) + b1l[...] + _dot_t(xb[...], w1r[...])
    h = jnp.maximum(h, 0.0)
    h_out[...] = h
    p2_out[...] = _dot_t(h, w2l[...])
    dinv_out[...] = jnp.broadcast_to(dinv, (BLK, IN))


_tc1 = pl.pallas_call(
    _tc1_body,
    grid=(R // BLK,),
    in_specs=[
        pl.BlockSpec((BLK, WE), lambda i: (i, 0)),
        pl.BlockSpec((BLK, WE), lambda i: (i, 0)),
        pl.BlockSpec((BLK, IN), lambda i: (i, 0)),
        pl.BlockSpec((HID, IN), lambda i: (0, 0)),
        pl.BlockSpec((1, HID), lambda i: (0, 0)),
        pl.BlockSpec((HID, IN), lambda i: (0, 0)),
        pl.BlockSpec((OUT, HID), lambda i: (0, 0)),
    ],
    out_specs=[
        pl.BlockSpec((BLK, HID), lambda i: (i, 0)),
        pl.BlockSpec((BLK, OUT), lambda i: (i, 0)),
        pl.BlockSpec((BLK, IN), lambda i: (i, 0)),
    ],
    out_shape=[
        jax.ShapeDtypeStruct((R, HID), jnp.float32),
        jax.ShapeDtypeStruct((R, OUT), jnp.float32),
        jax.ShapeDtypeStruct((R, IN), jnp.float32),
    ],
)


def _tc2_body(q0, q1, dinvf, hb, w2r, b2l, o_out):
    mean2 = (q0[...] + q1[...]) * dinvf[...]
    o_out[...] = mean2 + b2l[...] + _dot_t(hb[...], w2r[...])


_tc2 = pl.pallas_call(
    _tc2_body,
    grid=(R // BLK,),
    in_specs=[
        pl.BlockSpec((BLK, OUT), lambda i: (i, 0)),
        pl.BlockSpec((BLK, OUT), lambda i: (i, 0)),
        pl.BlockSpec((BLK, IN), lambda i: (i, 0)),
        pl.BlockSpec((BLK, HID), lambda i: (i, 0)),
        pl.BlockSpec((OUT, HID), lambda i: (0, 0)),
        pl.BlockSpec((1, OUT), lambda i: (0, 0)),
    ],
    out_specs=pl.BlockSpec((BLK, OUT), lambda i: (i, 0)),
    out_shape=jax.ShapeDtypeStruct((R, OUT), jnp.float32),
)


def kernel(x, edge, W1l, b1l, W1r, W2l, b2l, W2r):
    src = edge[0].astype(jnp.int32)
    dst = edge[1].astype(jnp.int32)
    srcp = jnp.concatenate(
        [src, jnp.zeros((EPAD - E,), jnp.int32)]).reshape(NW, NCHUNK, CHUNK)
    dstp = jnp.concatenate(
        [dst, jnp.full((EPAD - E,), N, jnp.int32)]).reshape(NW, NCHUNK, CHUNK)

    xe = jnp.pad(
        jnp.concatenate(
            [x, jnp.ones((N, 1), jnp.float32),
             jnp.zeros((N, WE - IN - 1), jnp.float32)], axis=1),
        ((0, R - N), (0, 0)))
    xp = jnp.pad(x, ((0, R - N), (0, 0)))
    z_we = jnp.zeros((R, WE), jnp.float32)
    z_out = jnp.zeros((R, OUT), jnp.float32)

    P = _sc_agg_l1(xe, srcp, dstp, z_we)
    h, p2, dinvf = _tc1(P[0], P[1], xp, W1l, b1l.reshape(1, HID), W1r, W2l)
    Q = _sc_agg_l2(p2, srcp, dstp, z_out)
    out = _tc2(Q[0], Q[1], dinvf, h, W2r, b2l.reshape(1, OUT))
    return out[:N]


# speedup vs baseline: 4.4745x; 4.4745x over previous
import functools

import jax
import jax.numpy as jnp
from jax import lax
from jax.experimental import pallas as pl
from jax.experimental.pallas import tpu as pltpu
from jax.experimental.pallas import tpu_sc as plsc

N = 10000
E = 320000
IN = 128
HID = 256
OUT = 128

BLK = 128
R = 10112
NW = 32
CHUNK = 128
NCHUNK = 79
EW = NCHUNK * CHUNK
EPAD = NW * EW
RPT = R // 16
BROWS = 80
TAIL = 72
DW = 16


def _make_sc_agg():
    mesh = plsc.VectorSubcoreMesh(core_axis_name="c", subcore_axis_name="s")

    @functools.partial(
        pl.kernel,
        mesh=mesh,
        out_type=jax.ShapeDtypeStruct((2, R, IN), jnp.float32),
        scratch_types=[
            pltpu.VMEM((CHUNK,), jnp.int32),
            pltpu.VMEM((CHUNK,), jnp.int32),
            pltpu.VMEM((CHUNK, IN), jnp.float32),
            pltpu.VMEM_SHARED((R, IN), jnp.float32),
            pltpu.SemaphoreType.DMA,
        ],
    )
    def body(rows_hbm, src_hbm, dst_hbm, zeros_hbm,
             out_hbm, src_i, dst_i, rows_v, acc, sem):
        c = lax.axis_index("c")
        s = lax.axis_index("s")
        wid = c * 16 + s
        ebase = wid * EW
        bounce = rows_v.at[pl.ds(0, BROWS)]
        pltpu.sync_copy(zeros_hbm, bounce)
        for k in range(7):
            pltpu.sync_copy(bounce, acc.at[pl.ds(s * RPT + k * BROWS, BROWS)])
        pltpu.sync_copy(bounce.at[pl.ds(0, TAIL)],
                        acc.at[pl.ds(s * RPT + 7 * BROWS, TAIL)])
        plsc.subcore_barrier()

        def ebody(i, carry):
            pltpu.sync_copy(src_hbm.at[pl.ds(ebase + i * CHUNK, CHUNK)], src_i)
            pltpu.sync_copy(dst_hbm.at[pl.ds(ebase + i * CHUNK, CHUNK)], dst_i)
            pltpu.async_copy(rows_hbm.at[src_i], rows_v, sem).wait()
            pltpu.sync_copy(rows_v, acc.at[dst_i], add=True)
            return carry

        lax.fori_loop(0, NCHUNK, ebody, 0)
        plsc.subcore_barrier()
        for k in range(8):
            r0 = s * RPT + k * BROWS
            nr = BROWS if k < 7 else TAIL
            pltpu.sync_copy(acc.at[pl.ds(r0, nr)], bounce.at[pl.ds(0, nr)])
            pltpu.sync_copy(bounce.at[pl.ds(0, nr)],
                            out_hbm.at[c, pl.ds(r0, nr)])

    return body


def _make_sc_deg():
    mesh = plsc.VectorSubcoreMesh(core_axis_name="c", subcore_axis_name="s")

    @functools.partial(
        pl.kernel,
        mesh=mesh,
        out_type=jax.ShapeDtypeStruct((2, R, DW), jnp.float32),
        scratch_types=[
            pltpu.VMEM((CHUNK,), jnp.int32),
            pltpu.VMEM((CHUNK, DW), jnp.float32),
            pltpu.VMEM_SHARED((R, DW), jnp.float32),
        ],
    )
    def body(dst_hbm, ones16_hbm, deg_hbm, dst_i, ones_v, acc_deg):
        c = lax.axis_index("c")
        s = lax.axis_index("s")
        wid = c * 16 + s
        ebase = wid * EW
        bounce = ones_v.at[pl.ds(0, BROWS)]
        pltpu.sync_copy(ones16_hbm.at[pl.ds(CHUNK, BROWS)], bounce)
        for k in range(7):
            pltpu.sync_copy(bounce,
                            acc_deg.at[pl.ds(s * RPT + k * BROWS, BROWS)])
        pltpu.sync_copy(bounce.at[pl.ds(0, TAIL)],
                        acc_deg.at[pl.ds(s * RPT + 7 * BROWS, TAIL)])
        pltpu.sync_copy(ones16_hbm.at[pl.ds(0, CHUNK)], ones_v)
        plsc.subcore_barrier()

        def ebody(i, carry):
            pltpu.sync_copy(dst_hbm.at[pl.ds(ebase + i * CHUNK, CHUNK)], dst_i)
            pltpu.sync_copy(ones_v, acc_deg.at[dst_i], add=True)
            return carry

        lax.fori_loop(0, NCHUNK, ebody, 0)
        plsc.subcore_barrier()
        for k in range(8):
            r0 = s * RPT + k * BROWS
            nr = BROWS if k < 7 else TAIL
            pltpu.sync_copy(acc_deg.at[pl.ds(r0, nr)], bounce.at[pl.ds(0, nr)])
            pltpu.sync_copy(bounce.at[pl.ds(0, nr)],
                            deg_hbm.at[c, pl.ds(r0, nr)])

    return body


_sc_agg = _make_sc_agg()
_sc_deg = _make_sc_deg()


def _dot_t(a, b):
    return lax.dot_general(a, b, (((1,), (1,)), ((), ())),
                           preferred_element_type=jnp.float32)


def _tc1_body(p0, p1, pd0, pd1, xb, w1l, b1l, w1r, w2l,
              h_out, p2_out, dinv_out):
    agg = p0[...] + p1[...]
    deg = pd0[:, :1] + pd1[:, :1]
    dinv = 1.0 / jnp.maximum(deg, 1.0)
    mean = agg * dinv
    h = _dot_t(mean, w1l[...]) + b1l[...] + _dot_t(xb[...], w1r[...])
    h = jnp.maximum(h, 0.0)
    h_out[...] = h
    p2_out[...] = _dot_t(h, w2l[...])
    dinv_out[...] = jnp.broadcast_to(dinv, (BLK, IN))


_tc1 = pl.pallas_call(
    _tc1_body,
    grid=(R // BLK,),
    in_specs=[
        pl.BlockSpec((BLK, IN), lambda i: (i, 0)),
        pl.BlockSpec((BLK, IN), lambda i: (i, 0)),
        pl.BlockSpec((BLK, DW), lambda i: (i, 0)),
        pl.BlockSpec((BLK, DW), lambda i: (i, 0)),
        pl.BlockSpec((BLK, IN), lambda i: (i, 0)),
        pl.BlockSpec((HID, IN), lambda i: (0, 0)),
        pl.BlockSpec((1, HID), lambda i: (0, 0)),
        pl.BlockSpec((HID, IN), lambda i: (0, 0)),
        pl.BlockSpec((OUT, HID), lambda i: (0, 0)),
    ],
    out_specs=[
        pl.BlockSpec((BLK, HID), lambda i: (i, 0)),
        pl.BlockSpec((BLK, OUT), lambda i: (i, 0)),
        pl.BlockSpec((BLK, IN), lambda i: (i, 0)),
    ],
    out_shape=[
        jax.ShapeDtypeStruct((R, HID), jnp.float32),
        jax.ShapeDtypeStruct((R, OUT), jnp.float32),
        jax.ShapeDtypeStruct((R, IN), jnp.float32),
    ],
)


def _tc2_body(q0, q1, dinvf, hb, w2r, b2l, o_out):
    mean2 = (q0[...] + q1[...]) * dinvf[...]
    o_out[...] = mean2 + b2l[...] + _dot_t(hb[...], w2r[...])


_tc2 = pl.pallas_call(
    _tc2_body,
    grid=(R // BLK,),
    in_specs=[
        pl.BlockSpec((BLK, OUT), lambda i: (i, 0)),
        pl.BlockSpec((BLK, OUT), lambda i: (i, 0)),
        pl.BlockSpec((BLK, IN), lambda i: (i, 0)),
        pl.BlockSpec((BLK, HID), lambda i: (i, 0)),
        pl.BlockSpec((OUT, HID), lambda i: (0, 0)),
        pl.BlockSpec((1, OUT), lambda i: (0, 0)),
    ],
    out_specs=pl.BlockSpec((BLK, OUT), lambda i: (i, 0)),
    out_shape=jax.ShapeDtypeStruct((R, OUT), jnp.float32),
)


def kernel(x, edge, W1l, b1l, W1r, W2l, b2l, W2r):
    src = edge[0].astype(jnp.int32)
    dst = edge[1].astype(jnp.int32)
    srcp = jnp.concatenate([src, jnp.zeros((EPAD - E,), jnp.int32)])
    dstp = jnp.concatenate([dst, jnp.full((EPAD - E,), N, jnp.int32)])

    xp = jnp.pad(x, ((0, R - N), (0, 0)))
    zeros = jnp.zeros((BROWS, IN), jnp.float32)
    ones8 = jnp.concatenate([jnp.ones((CHUNK, DW), jnp.float32),
                             jnp.zeros((BROWS, DW), jnp.float32)])

    P = _sc_agg(xp, srcp, dstp, zeros)
    PD = _sc_deg(dstp, ones8)
    h, p2, dinvf = _tc1(P[0], P[1], PD[0], PD[1], xp,
                        W1l, b1l.reshape(1, HID), W1r, W2l)
    Q = _sc_agg(p2, srcp, dstp, zeros)
    out = _tc2(Q[0], Q[1], dinvf, h, W2r, b2l.reshape(1, OUT))
    return out[:N]
